# baseline (speedup 1.0000x reference)
import functools

import jax
import jax.numpy as jnp
from jax import lax
from jax.experimental import pallas as pl
from jax.experimental.pallas import tpu as pltpu
from jax.experimental.pallas import tpu_sc as plsc

_B, _F, _V, _D = 16384, 26, 40000, 16
_T = _F * _V
_FLO = 14
_FHI = _F - _FLO
_TLO = _FLO * _V
_THI = _FHI * _V
_NC, _NS, _L = 2, 16, 16
_NW = _NC * _NS
_BPW = _B // _NW
_GPR = _BPW // _L

_CBLK = 80000

_mesh = plsc.VectorSubcoreMesh(core_axis_name="c", subcore_axis_name="s")


def _make_pcalc(nrows, col_off):
    def body(lb_ref, wt_ref, lwt_ref, mcol_ref, p_ref):
        i = pl.program_id(0)
        lwb = jnp.broadcast_to(lwt_ref[...], (_D, _CBLK))
        s = jnp.sum(wt_ref[...] * lwb, axis=0)
        p_ref[pl.ds(i * _CBLK, _CBLK)] = (
            s * mcol_ref[pl.ds(i * _CBLK, _CBLK)] + lb_ref[0] * (1.0 / _F)
        )

    nblk = nrows // _CBLK
    return pl.pallas_call(
        body,
        grid=(nblk,),
        in_specs=[
            pl.BlockSpec(memory_space=pltpu.SMEM),
            pl.BlockSpec((_D, _CBLK), lambda i: (0, col_off // _CBLK + i)),
            pl.BlockSpec((_D, 1), lambda i: (0, 0)),
            pl.BlockSpec((nrows,), lambda i: (0,)),
        ],
        out_specs=pl.BlockSpec((nrows,), lambda i: (0,)),
        out_shape=jax.ShapeDtypeStruct((nrows,), jnp.float32),
    )


_pcalc_lo = _make_pcalc(_TLO, 0)
_pcalc_hi = _make_pcalc(_THI, _TLO)


def _make_pool(nf, add_partial):
    scratch = [
        pltpu.VMEM((nf, _BPW), jnp.int32),
        pltpu.VMEM((nf, _BPW), jnp.int32),
        pltpu.VMEM((nf, _BPW), jnp.float32),
        pltpu.VMEM((_BPW,), jnp.float32),
        pltpu.SemaphoreType.DMA,
    ]
    if add_partial:
        scratch.insert(3, pltpu.VMEM((_BPW,), jnp.float32))

    @functools.partial(
        pl.kernel,
        mesh=_mesh,
        compiler_params=pltpu.CompilerParams(use_tc_tiling_on_sc=False),
        out_type=jax.ShapeDtypeStruct((_B,), jnp.float32),
        scratch_types=scratch,
    )
    def pool(*args):
        if add_partial:
            xt_hbm, part_hbm, p_hbm, o_hbm, xtv, idxq, sv, pv, hv, sem = args
        else:
            xt_hbm, p_hbm, o_hbm, xtv, idxq, sv, hv, sem = args
            part_hbm = pv = None
        wid = lax.axis_index("s") * _NC + lax.axis_index("c")
        base = wid * _BPW
        stages = [
            pltpu.async_copy(
                xt_hbm.at[pl.ds(f * _B + base, _BPW)], xtv.at[f], sem
            )
            for f in range(nf)
        ]
        if add_partial:
            stages.append(
                pltpu.async_copy(part_hbm.at[pl.ds(base, _BPW)], pv, sem)
            )
        for cp in stages:
            cp.wait()

        for f in range(nf):
            for g in range(_GPR):
                idxq[f, pl.ds(g * _L, _L)] = (
                    xtv[f, pl.ds(g * _L, _L)] + f * _V
                )
        copies = [
            pltpu.async_copy(p_hbm.at[idxq.at[f]], sv.at[f], sem)
            for f in range(nf)
        ]
        copies[0].wait()
        if add_partial:
            acc = [
                pv[pl.ds(g * _L, _L)] + sv[0, pl.ds(g * _L, _L)]
                for g in range(_GPR)
            ]
        else:
            acc = [sv[0, pl.ds(g * _L, _L)] for g in range(_GPR)]
        for f in range(1, nf):
            copies[f].wait()
            acc = [a + sv[f, pl.ds(g * _L, _L)] for g, a in enumerate(acc)]
        for g in range(_GPR):
            hv[pl.ds(g * _L, _L)] = acc[g]

        pltpu.sync_copy(hv, o_hbm.at[pl.ds(base, _BPW)])

    return pool


_pool_lo = _make_pool(_FLO, False)
_pool_hi = _make_pool(_FHI, True)


def kernel(x, field_mask, new_field_mask, w, lin_w, lin_b):
    xt = jnp.transpose(x.astype(jnp.int32)).reshape((_F * _B,))
    xt_lo = xt[: _FLO * _B]
    xt_hi = xt[_FLO * _B:]
    wt = jnp.transpose(w)
    lwt = jnp.transpose(lin_w)
    mf = field_mask.astype(jnp.float32)
    p_lo = _pcalc_lo(lin_b, wt, lwt, jnp.repeat(mf[:_FLO], _V))
    p_hi = _pcalc_hi(lin_b, wt, lwt, jnp.repeat(mf[_FLO:], _V))
    o_part = _pool_lo(xt_lo, p_lo)
    o = _pool_hi(xt_hi, o_part, p_hi)
    return o.reshape(_B, 1)


# speedup vs baseline: 1.0107x; 1.0107x over previous
import functools

import jax
import jax.numpy as jnp
from jax import lax
from jax.experimental import pallas as pl
from jax.experimental.pallas import tpu as pltpu
from jax.experimental.pallas import tpu_sc as plsc

_B, _F, _V, _D = 16384, 26, 40000, 16
_T = _F * _V
_NC, _NS, _L = 2, 16, 16
_NW = _NC * _NS
_BPW = _B // _NW
_CHUNK = 512
_NCHUNK = _BPW // _CHUNK
_GPR = _CHUNK // _L

_CBLK = 208000

_mesh = plsc.VectorSubcoreMesh(core_axis_name="c", subcore_axis_name="s")


def _pcalc_body(lb_ref, wt_ref, lwt_ref, mcol_ref, p_ref):
    i = pl.program_id(0)
    lwb = jnp.broadcast_to(lwt_ref[...], (_D, _CBLK))
    s = jnp.sum(wt_ref[...] * lwb, axis=0)
    p_ref[pl.ds(i * _CBLK, _CBLK)] = (
        s * mcol_ref[pl.ds(i * _CBLK, _CBLK)] + lb_ref[0] * (1.0 / _F)
    )


_pcalc = pl.pallas_call(
    _pcalc_body,
    grid=(_T // _CBLK,),
    in_specs=[
        pl.BlockSpec(memory_space=pltpu.SMEM),
        pl.BlockSpec((_D, _CBLK), lambda i: (0, i)),
        pl.BlockSpec((_D, 1), lambda i: (0, 0)),
        pl.BlockSpec((_T,), lambda i: (0,)),
    ],
    out_specs=pl.BlockSpec((_T,), lambda i: (0,)),
    out_shape=jax.ShapeDtypeStruct((_T,), jnp.float32),
)


@functools.partial(
    pl.kernel,
    mesh=_mesh,
    compiler_params=pltpu.CompilerParams(use_tc_tiling_on_sc=False),
    out_type=jax.ShapeDtypeStruct((_B,), jnp.float32),
    scratch_types=[
        pltpu.VMEM((_F, _BPW), jnp.int32),
        pltpu.VMEM((_F, _CHUNK), jnp.int32),
        pltpu.VMEM((_F, _CHUNK), jnp.float32),
        pltpu.VMEM((_CHUNK,), jnp.float32),
        pltpu.SemaphoreType.DMA,
    ],
)
def _pool(xt_hbm, p_hbm, o_hbm, xtv, idxq, sv, hv, sem):
    wid = lax.axis_index("s") * _NC + lax.axis_index("c")
    stages = [
        pltpu.async_copy(
            xt_hbm.at[pl.ds(f * _B + wid * _BPW, _BPW)], xtv.at[f], sem
        )
        for f in range(_F)
    ]
    for cp in stages:
        cp.wait()

    def chunk_body(k, carry):
        base = wid * _BPW + k * _CHUNK
        for f in range(_F):
            for g in range(_GPR):
                idxq[f, pl.ds(g * _L, _L)] = (
                    xtv[f, pl.ds(k * _CHUNK + g * _L, _L)] + f * _V
                )
        copies = [
            pltpu.async_copy(p_hbm.at[idxq.at[f]], sv.at[f], sem)
            for f in range(_F)
        ]
        copies[0].wait()
        acc = [sv[0, pl.ds(g * _L, _L)] for g in range(_GPR)]
        for f in range(1, _F):
            copies[f].wait()
            acc = [a + sv[f, pl.ds(g * _L, _L)] for g, a in enumerate(acc)]
        for g in range(_GPR):
            hv[pl.ds(g * _L, _L)] = acc[g]

        pltpu.sync_copy(hv, o_hbm.at[pl.ds(base, _CHUNK)])
        return carry

    lax.fori_loop(0, _NCHUNK, chunk_body, 0)


def kernel(x, field_mask, new_field_mask, w, lin_w, lin_b):
    xt = jnp.transpose(x.astype(jnp.int32)).reshape((_F * _B,))
    wt = jnp.transpose(w)
    lwt = jnp.transpose(lin_w)
    mcol = jnp.repeat(field_mask.astype(jnp.float32), _V)
    p = _pcalc(lin_b, wt, lwt, mcol)
    o = _pool(xt, p)
    return o.reshape(_B, 1)
